# baseline (speedup 1.0000x reference)
import functools

import jax
import jax.numpy as jnp
from jax import lax
from jax.experimental import pallas as pl
from jax.experimental.pallas import tpu as pltpu
from jax.experimental.pallas import tpu_sc as plsc

N = 10000
E = 320000
D = 128
NC = 2
NS = 16
NW = NC * NS
CH = 128
EPW = E // NW
CHK0 = 117
CHK1 = 40
NCOL = max(CHK0, CHK1)
NPAD = 10240
RPT = NPAD // NS
ZC = RPT // CH
RB = 512
GRID = NPAD // RB

_mesh = plsc.VectorSubcoreMesh(core_axis_name="c", subcore_axis_name="s",
                               num_cores=NC, num_subcores=NS)


def _deg_body(dst_hbm, out_hbm, idx_v, acc_v, sem):
    c = lax.axis_index("c")
    s = lax.axis_index("s")
    wid = c * NS + s
    ones = jnp.ones((16,), jnp.float32)

    def _z(i, _):
        acc_v[pl.ds(i * 16, 16)] = jnp.zeros((16,), jnp.float32)
        return 0

    lax.fori_loop(0, NPAD // 16, _z, 0)
    pltpu.sync_copy(dst_hbm.at[wid], idx_v)

    def _step(g, _):
        idx = idx_v[g // (CH // 16), pl.ds((g % (CH // 16)) * 16, 16)]
        plsc.addupdate_scatter(acc_v, [idx], ones)
        return 0

    lax.fori_loop(0, NCOL * (CH // 16), _step, 0)
    pltpu.sync_copy(acc_v, out_hbm.at[wid])


_deg_call = pl.kernel(
    _deg_body,
    out_type=jax.ShapeDtypeStruct((NW, NPAD), jnp.float32),
    mesh=_mesh,
    scratch_types=[
        pltpu.VMEM((NCOL, CH), jnp.int32),
        pltpu.VMEM((NPAD,), jnp.float32),
        pltpu.SemaphoreType.DMA,
    ],
    compiler_params=pltpu.CompilerParams(needs_layout_passes=False),
)


def _agg_body(h_hbm, ei_hbm, out_hbm, idx_v, rows_v, acc_sh, sem):
    c = lax.axis_index("c")
    s = lax.axis_index("s")
    wid = c * NS + s

    def _zrow(i, _):
        def _z16(k, _):
            rows_v[i, pl.ds(k * 16, 16)] = jnp.zeros((16,), jnp.float32)
            return 0
        lax.fori_loop(0, D // 16, _z16, 0)
        return 0

    lax.fori_loop(0, CH, _zrow, 0)
    for k in range(ZC):
        pltpu.sync_copy(rows_v, acc_sh.at[pl.ds(s * RPT + k * CH, CH)])
    plsc.subcore_barrier()

    nch = jnp.where(c == 0, CHK0, CHK1)

    def _step(j, _):
        pltpu.sync_copy(ei_hbm.at[wid, j], idx_v)
        pltpu.async_copy(h_hbm.at[idx_v.at[0]], rows_v, sem).wait()
        pltpu.sync_copy(rows_v, acc_sh.at[idx_v.at[1]], add=True)
        return 0

    lax.fori_loop(0, nch, _step, 0)
    plsc.subcore_barrier()
    pltpu.sync_copy(acc_sh.at[pl.ds(s * RPT, RPT)],
                    out_hbm.at[c, pl.ds(s * RPT, RPT)])


_agg_call = pl.kernel(
    _agg_body,
    out_type=jax.ShapeDtypeStruct((NC, NPAD, D), jnp.float32),
    mesh=_mesh,
    scratch_types=[
        pltpu.VMEM((2, CH), jnp.int32),
        pltpu.VMEM((CH, D), jnp.float32),
        pltpu.VMEM_SHARED((NPAD, D), jnp.float32),
        pltpu.SemaphoreType.DMA,
    ],
    compiler_params=pltpu.CompilerParams(needs_layout_passes=False),
)


def _dis_from(deg_blk):
    deg = jnp.sum(deg_blk, axis=0) + 1.0
    return lax.rsqrt(deg).reshape(RB, 1)


def _tc1_body(x_ref, w_ref, deg_ref, o_ref):
    dis = _dis_from(deg_ref[...])
    o_ref[...] = jnp.dot(x_ref[...], w_ref[...],
                         preferred_element_type=jnp.float32) * dis


def _tc2_body(acc_ref, hp_ref, deg_ref, b_ref, w_ref, o_ref):
    dis = _dis_from(deg_ref[...])
    agg = acc_ref[0] + acc_ref[1] + hp_ref[...]
    out1 = jnp.maximum(dis * agg + b_ref[...], 0.0)
    o_ref[...] = jnp.dot(out1, w_ref[...],
                         preferred_element_type=jnp.float32) * dis


def _tc3_body(acc_ref, hp_ref, deg_ref, b_ref, o_ref):
    dis = _dis_from(deg_ref[...])
    o_ref[...] = dis * (acc_ref[0] + acc_ref[1] + hp_ref[...]) + b_ref[...]


_deg_spec = pl.BlockSpec((NW, RB), lambda i: (0, i))
_acc_spec = pl.BlockSpec((2, RB, D), lambda i: (0, i, 0))
_row_spec = pl.BlockSpec((RB, D), lambda i: (i, 0))
_mat_spec = pl.BlockSpec((D, D), lambda i: (0, 0))
_vec_spec = pl.BlockSpec((1, D), lambda i: (0, 0))
_f32 = functools.partial(jax.ShapeDtypeStruct, dtype=jnp.float32)

_tc1_call = pl.pallas_call(
    _tc1_body, grid=(GRID,),
    in_specs=[_row_spec, _mat_spec, _deg_spec],
    out_specs=_row_spec, out_shape=_f32(shape=(NPAD, D)))

_tc2_call = pl.pallas_call(
    _tc2_body, grid=(GRID,),
    in_specs=[_acc_spec, _row_spec, _deg_spec, _vec_spec, _mat_spec],
    out_specs=_row_spec, out_shape=_f32(shape=(NPAD, D)))

_tc3_call = pl.pallas_call(
    _tc3_body, grid=(GRID,),
    in_specs=[_acc_spec, _row_spec, _deg_spec, _vec_spec],
    out_specs=_row_spec, out_shape=_f32(shape=(NPAD, D)))


@jax.jit
def _run(x, ei, dst, W1, b1, W2, b2):
    degacc = _deg_call(dst)
    hp1 = _tc1_call(x, W1, degacc)
    acc1 = _agg_call(hp1, ei)
    hp2 = _tc2_call(acc1, hp1, degacc, b1.reshape(1, D), W2)
    acc2 = _agg_call(hp2, ei)
    return _tc3_call(acc2, hp2, degacc, b2.reshape(1, D))


def kernel(x, edge_index, W1, b1, W2, b2):
    x = jnp.concatenate([x, jnp.zeros((NPAD - N, D), jnp.float32)])
    src = edge_index[0].astype(jnp.int32)
    dst = edge_index[1].astype(jnp.int32)
    epad = NS * (CHK0 + CHK1) * CH
    src = jnp.concatenate([src, jnp.zeros((epad - E,), jnp.int32)])
    pad_d = N + (jnp.arange(epad - E, dtype=jnp.int32) % (NPAD - N))
    dst = jnp.concatenate([dst, pad_d])

    def _part(flat):
        e0 = NS * CHK0 * CH
        p0 = flat[:e0].reshape(NS, CHK0, CH)
        p1 = flat[e0:].reshape(NS, CHK1, CH)
        pv = jnp.broadcast_to(
            (N + jnp.arange(CH, dtype=jnp.int32) % (NPAD - N))[None, None, :],
            (NS, NCOL - min(CHK0, CHK1), CH))
        if CHK0 < CHK1:
            p0 = jnp.concatenate([p0, pv[:, :NCOL - CHK0]], axis=1)
        elif CHK1 < CHK0:
            p1 = jnp.concatenate([p1, pv[:, :NCOL - CHK1]], axis=1)
        return jnp.concatenate([p0, p1], axis=0)

    src = _part(src)
    dst = _part(dst)
    ei = jnp.stack([src, dst], axis=2)
    return _run(x, ei, dst, W1, b1, W2, b2)[:N]


# speedup vs baseline: 1.0501x; 1.0501x over previous
import functools

import jax
import jax.numpy as jnp
from jax import lax
from jax.experimental import pallas as pl
from jax.experimental.pallas import tpu as pltpu
from jax.experimental.pallas import tpu_sc as plsc

N = 10000
E = 320000
D = 128
NC = 2
NS = 16
NW = NC * NS
CH = 128
EPW = E // NW
CHK0 = 110
CHK1 = 47
NCOL = max(CHK0, CHK1)
NPAD = 10240
RPT = NPAD // NS
ZC = RPT // CH
RB = 512
GRID = NPAD // RB

_mesh = plsc.VectorSubcoreMesh(core_axis_name="c", subcore_axis_name="s",
                               num_cores=NC, num_subcores=NS)


def _deg_body(dst_hbm, out_hbm, idx_v, acc_v, sem):
    c = lax.axis_index("c")
    s = lax.axis_index("s")
    wid = c * NS + s
    ones = jnp.ones((16,), jnp.float32)

    def _z(i, _):
        acc_v[pl.ds(i * 16, 16)] = jnp.zeros((16,), jnp.float32)
        return 0

    lax.fori_loop(0, NPAD // 16, _z, 0)
    pltpu.sync_copy(dst_hbm.at[wid], idx_v)

    def _step(g, _):
        idx = idx_v[g // (CH // 16), pl.ds((g % (CH // 16)) * 16, 16)]
        plsc.addupdate_scatter(acc_v, [idx], ones)
        return 0

    lax.fori_loop(0, NCOL * (CH // 16), _step, 0)
    pltpu.sync_copy(acc_v, out_hbm.at[wid])


_deg_call = pl.kernel(
    _deg_body,
    out_type=jax.ShapeDtypeStruct((NW, NPAD), jnp.float32),
    mesh=_mesh,
    scratch_types=[
        pltpu.VMEM((NCOL, CH), jnp.int32),
        pltpu.VMEM((NPAD,), jnp.float32),
        pltpu.SemaphoreType.DMA,
    ],
    compiler_params=pltpu.CompilerParams(needs_layout_passes=False),
)


def _agg_body(h_hbm, ei_hbm, out_hbm, idx_v, rows_v, acc_sh, sem):
    c = lax.axis_index("c")
    s = lax.axis_index("s")
    wid = c * NS + s

    def _zrow(i, _):
        def _z16(k, _):
            rows_v[i, pl.ds(k * 16, 16)] = jnp.zeros((16,), jnp.float32)
            return 0
        lax.fori_loop(0, D // 16, _z16, 0)
        return 0

    lax.fori_loop(0, CH, _zrow, 0)
    for k in range(ZC):
        pltpu.sync_copy(rows_v, acc_sh.at[pl.ds(s * RPT + k * CH, CH)])
    plsc.subcore_barrier()

    nch = jnp.where(c == 0, CHK0, CHK1)

    def _step(j, _):
        pltpu.sync_copy(ei_hbm.at[wid, j], idx_v)
        pltpu.async_copy(h_hbm.at[idx_v.at[0]], rows_v, sem).wait()
        pltpu.sync_copy(rows_v, acc_sh.at[idx_v.at[1]], add=True)
        return 0

    lax.fori_loop(0, nch, _step, 0)
    plsc.subcore_barrier()
    pltpu.sync_copy(acc_sh.at[pl.ds(s * RPT, RPT)],
                    out_hbm.at[c, pl.ds(s * RPT, RPT)])


_agg_call = pl.kernel(
    _agg_body,
    out_type=jax.ShapeDtypeStruct((NC, NPAD, D), jnp.float32),
    mesh=_mesh,
    scratch_types=[
        pltpu.VMEM((2, CH), jnp.int32),
        pltpu.VMEM((CH, D), jnp.float32),
        pltpu.VMEM_SHARED((NPAD, D), jnp.float32),
        pltpu.SemaphoreType.DMA,
    ],
    compiler_params=pltpu.CompilerParams(needs_layout_passes=False),
)


def _dis_from(deg_blk):
    deg = jnp.sum(deg_blk, axis=0) + 1.0
    return lax.rsqrt(deg).reshape(RB, 1)


def _tc1_body(x_ref, w_ref, deg_ref, o_ref):
    dis = _dis_from(deg_ref[...])
    o_ref[...] = jnp.dot(x_ref[...], w_ref[...],
                         preferred_element_type=jnp.float32) * dis


def _tc2_body(acc_ref, hp_ref, deg_ref, b_ref, w_ref, o_ref):
    dis = _dis_from(deg_ref[...])
    agg = acc_ref[0] + acc_ref[1] + hp_ref[...]
    out1 = jnp.maximum(dis * agg + b_ref[...], 0.0)
    o_ref[...] = jnp.dot(out1, w_ref[...],
                         preferred_element_type=jnp.float32) * dis


def _tc3_body(acc_ref, hp_ref, deg_ref, b_ref, o_ref):
    dis = _dis_from(deg_ref[...])
    o_ref[...] = dis * (acc_ref[0] + acc_ref[1] + hp_ref[...]) + b_ref[...]


_deg_spec = pl.BlockSpec((NW, RB), lambda i: (0, i))
_acc_spec = pl.BlockSpec((2, RB, D), lambda i: (0, i, 0))
_row_spec = pl.BlockSpec((RB, D), lambda i: (i, 0))
_mat_spec = pl.BlockSpec((D, D), lambda i: (0, 0))
_vec_spec = pl.BlockSpec((1, D), lambda i: (0, 0))
_f32 = functools.partial(jax.ShapeDtypeStruct, dtype=jnp.float32)

_tc1_call = pl.pallas_call(
    _tc1_body, grid=(GRID,),
    in_specs=[_row_spec, _mat_spec, _deg_spec],
    out_specs=_row_spec, out_shape=_f32(shape=(NPAD, D)))

_tc2_call = pl.pallas_call(
    _tc2_body, grid=(GRID,),
    in_specs=[_acc_spec, _row_spec, _deg_spec, _vec_spec, _mat_spec],
    out_specs=_row_spec, out_shape=_f32(shape=(NPAD, D)))

_tc3_call = pl.pallas_call(
    _tc3_body, grid=(GRID,),
    in_specs=[_acc_spec, _row_spec, _deg_spec, _vec_spec],
    out_specs=_row_spec, out_shape=_f32(shape=(NPAD, D)))


@jax.jit
def _run(x, ei, dst, W1, b1, W2, b2):
    degacc = _deg_call(dst)
    hp1 = _tc1_call(x, W1, degacc)
    acc1 = _agg_call(hp1, ei)
    hp2 = _tc2_call(acc1, hp1, degacc, b1.reshape(1, D), W2)
    acc2 = _agg_call(hp2, ei)
    return _tc3_call(acc2, hp2, degacc, b2.reshape(1, D))


def kernel(x, edge_index, W1, b1, W2, b2):
    x = jnp.concatenate([x, jnp.zeros((NPAD - N, D), jnp.float32)])
    src = edge_index[0].astype(jnp.int32)
    dst = edge_index[1].astype(jnp.int32)
    epad = NS * (CHK0 + CHK1) * CH
    src = jnp.concatenate([src, jnp.zeros((epad - E,), jnp.int32)])
    pad_d = N + (jnp.arange(epad - E, dtype=jnp.int32) % (NPAD - N))
    dst = jnp.concatenate([dst, pad_d])

    def _part(flat):
        e0 = NS * CHK0 * CH
        p0 = flat[:e0].reshape(NS, CHK0, CH)
        p1 = flat[e0:].reshape(NS, CHK1, CH)
        pv = jnp.broadcast_to(
            (N + jnp.arange(CH, dtype=jnp.int32) % (NPAD - N))[None, None, :],
            (NS, NCOL - min(CHK0, CHK1), CH))
        if CHK0 < CHK1:
            p0 = jnp.concatenate([p0, pv[:, :NCOL - CHK0]], axis=1)
        elif CHK1 < CHK0:
            p1 = jnp.concatenate([p1, pv[:, :NCOL - CHK1]], axis=1)
        return jnp.concatenate([p0, p1], axis=0)

    src = _part(src)
    dst = _part(dst)
    ei = jnp.stack([src, dst], axis=2)
    return _run(x, ei, dst, W1, b1, W2, b2)[:N]


# speedup vs baseline: 1.1037x; 1.0510x over previous
import functools

import jax
import jax.numpy as jnp
from jax import lax
from jax.experimental import pallas as pl
from jax.experimental.pallas import tpu as pltpu
from jax.experimental.pallas import tpu_sc as plsc

N = 10000
E = 320000
D = 128
NC = 2
NS = 16
NW = NC * NS
CH = 128
EPW = E // NW
CHK0 = 103
CHK1 = 54
NCOL = max(CHK0, CHK1)
NPAD = 10240
RPT = NPAD // NS
ZC = RPT // CH
RB = 512
GRID = NPAD // RB

_mesh = plsc.VectorSubcoreMesh(core_axis_name="c", subcore_axis_name="s",
                               num_cores=NC, num_subcores=NS)


def _deg_body(dst_hbm, out_hbm, idx_v, acc_v, sem):
    c = lax.axis_index("c")
    s = lax.axis_index("s")
    wid = c * NS + s
    ones = jnp.ones((16,), jnp.float32)

    def _z(i, _):
        acc_v[pl.ds(i * 16, 16)] = jnp.zeros((16,), jnp.float32)
        return 0

    lax.fori_loop(0, NPAD // 16, _z, 0)
    pltpu.sync_copy(dst_hbm.at[wid], idx_v)

    def _step(g, _):
        idx = idx_v[g // (CH // 16), pl.ds((g % (CH // 16)) * 16, 16)]
        plsc.addupdate_scatter(acc_v, [idx], ones)
        return 0

    lax.fori_loop(0, NCOL * (CH // 16), _step, 0)
    pltpu.sync_copy(acc_v, out_hbm.at[wid])


_deg_call = pl.kernel(
    _deg_body,
    out_type=jax.ShapeDtypeStruct((NW, NPAD), jnp.float32),
    mesh=_mesh,
    scratch_types=[
        pltpu.VMEM((NCOL, CH), jnp.int32),
        pltpu.VMEM((NPAD,), jnp.float32),
        pltpu.SemaphoreType.DMA,
    ],
    compiler_params=pltpu.CompilerParams(needs_layout_passes=False),
)


def _agg_body(h_hbm, ei_hbm, out_hbm, idx_v, rows_v, acc_sh, sem):
    c = lax.axis_index("c")
    s = lax.axis_index("s")
    wid = c * NS + s

    def _zrow(i, _):
        def _z16(k, _):
            rows_v[i, pl.ds(k * 16, 16)] = jnp.zeros((16,), jnp.float32)
            return 0
        lax.fori_loop(0, D // 16, _z16, 0)
        return 0

    lax.fori_loop(0, CH, _zrow, 0)
    for k in range(ZC):
        pltpu.sync_copy(rows_v, acc_sh.at[pl.ds(s * RPT + k * CH, CH)])
    plsc.subcore_barrier()

    nch = jnp.where(c == 0, CHK0, CHK1)

    def _step(j, _):
        pltpu.sync_copy(ei_hbm.at[wid, j], idx_v)
        pltpu.async_copy(h_hbm.at[idx_v.at[0]], rows_v, sem).wait()
        pltpu.sync_copy(rows_v, acc_sh.at[idx_v.at[1]], add=True)
        return 0

    lax.fori_loop(0, nch, _step, 0)
    plsc.subcore_barrier()
    pltpu.sync_copy(acc_sh.at[pl.ds(s * RPT, RPT)],
                    out_hbm.at[c, pl.ds(s * RPT, RPT)])


_agg_call = pl.kernel(
    _agg_body,
    out_type=jax.ShapeDtypeStruct((NC, NPAD, D), jnp.float32),
    mesh=_mesh,
    scratch_types=[
        pltpu.VMEM((2, CH), jnp.int32),
        pltpu.VMEM((CH, D), jnp.float32),
        pltpu.VMEM_SHARED((NPAD, D), jnp.float32),
        pltpu.SemaphoreType.DMA,
    ],
    compiler_params=pltpu.CompilerParams(needs_layout_passes=False),
)


def _dis_from(deg_blk):
    deg = jnp.sum(deg_blk, axis=0) + 1.0
    return lax.rsqrt(deg).reshape(RB, 1)


def _tc1_body(x_ref, w_ref, deg_ref, o_ref):
    dis = _dis_from(deg_ref[...])
    o_ref[...] = jnp.dot(x_ref[...], w_ref[...],
                         preferred_element_type=jnp.float32) * dis


def _tc2_body(acc_ref, hp_ref, deg_ref, b_ref, w_ref, o_ref):
    dis = _dis_from(deg_ref[...])
    agg = acc_ref[0] + acc_ref[1] + hp_ref[...]
    out1 = jnp.maximum(dis * agg + b_ref[...], 0.0)
    o_ref[...] = jnp.dot(out1, w_ref[...],
                         preferred_element_type=jnp.float32) * dis


def _tc3_body(acc_ref, hp_ref, deg_ref, b_ref, o_ref):
    dis = _dis_from(deg_ref[...])
    o_ref[...] = dis * (acc_ref[0] + acc_ref[1] + hp_ref[...]) + b_ref[...]


_deg_spec = pl.BlockSpec((NW, RB), lambda i: (0, i))
_acc_spec = pl.BlockSpec((2, RB, D), lambda i: (0, i, 0))
_row_spec = pl.BlockSpec((RB, D), lambda i: (i, 0))
_mat_spec = pl.BlockSpec((D, D), lambda i: (0, 0))
_vec_spec = pl.BlockSpec((1, D), lambda i: (0, 0))
_f32 = functools.partial(jax.ShapeDtypeStruct, dtype=jnp.float32)

_tc1_call = pl.pallas_call(
    _tc1_body, grid=(GRID,),
    in_specs=[_row_spec, _mat_spec, _deg_spec],
    out_specs=_row_spec, out_shape=_f32(shape=(NPAD, D)))

_tc2_call = pl.pallas_call(
    _tc2_body, grid=(GRID,),
    in_specs=[_acc_spec, _row_spec, _deg_spec, _vec_spec, _mat_spec],
    out_specs=_row_spec, out_shape=_f32(shape=(NPAD, D)))

_tc3_call = pl.pallas_call(
    _tc3_body, grid=(GRID,),
    in_specs=[_acc_spec, _row_spec, _deg_spec, _vec_spec],
    out_specs=_row_spec, out_shape=_f32(shape=(NPAD, D)))


@jax.jit
def _run(x, ei, dst, W1, b1, W2, b2):
    degacc = _deg_call(dst)
    hp1 = _tc1_call(x, W1, degacc)
    acc1 = _agg_call(hp1, ei)
    hp2 = _tc2_call(acc1, hp1, degacc, b1.reshape(1, D), W2)
    acc2 = _agg_call(hp2, ei)
    return _tc3_call(acc2, hp2, degacc, b2.reshape(1, D))


def kernel(x, edge_index, W1, b1, W2, b2):
    x = jnp.concatenate([x, jnp.zeros((NPAD - N, D), jnp.float32)])
    src = edge_index[0].astype(jnp.int32)
    dst = edge_index[1].astype(jnp.int32)
    epad = NS * (CHK0 + CHK1) * CH
    src = jnp.concatenate([src, jnp.zeros((epad - E,), jnp.int32)])
    pad_d = N + (jnp.arange(epad - E, dtype=jnp.int32) % (NPAD - N))
    dst = jnp.concatenate([dst, pad_d])

    def _part(flat):
        e0 = NS * CHK0 * CH
        p0 = flat[:e0].reshape(NS, CHK0, CH)
        p1 = flat[e0:].reshape(NS, CHK1, CH)
        pv = jnp.broadcast_to(
            (N + jnp.arange(CH, dtype=jnp.int32) % (NPAD - N))[None, None, :],
            (NS, NCOL - min(CHK0, CHK1), CH))
        if CHK0 < CHK1:
            p0 = jnp.concatenate([p0, pv[:, :NCOL - CHK0]], axis=1)
        elif CHK1 < CHK0:
            p1 = jnp.concatenate([p1, pv[:, :NCOL - CHK1]], axis=1)
        return jnp.concatenate([p0, p1], axis=0)

    src = _part(src)
    dst = _part(dst)
    ei = jnp.stack([src, dst], axis=2)
    return _run(x, ei, dst, W1, b1, W2, b2)[:N]
